# Initial kernel scaffold; baseline (speedup 1.0000x reference)
import jax
import jax.numpy as jnp
from jax.experimental import pallas as pl


def kernel(x, edge_indices, W, att, bias):
    raise NotImplementedError("write your pallas kernel here")


# speedup vs baseline: 3.8232x; 3.8232x over previous
import jax
import jax.numpy as jnp
from jax import lax
from jax.experimental import pallas as pl
from jax.experimental.pallas import tpu as pltpu
from jax.experimental.pallas import tpu_sc as plsc

F32 = jnp.float32

N_NODES = 50000
N_EDGES = 800000
D = 64
CPC = 8
NSLOT = D // CPC
NPH = NSLOT // 2
NP = 50176
NSUB = 16
RPT = NP // NSUB
CH = 80
CPT = N_EDGES // (CH * NSUB)


def _pre_body(x_ref, w_ref, a1_ref, a2_ref, h8_ref, s1_ref, s2_ref):
    xb = x_ref[...]
    h = lax.dot_general(xb, w_ref[...], (((1,), (0,)), ((), ())),
                        precision=lax.Precision.HIGHEST,
                        preferred_element_type=F32)
    for k in range(NSLOT):
        h8_ref[k] = h[:, k * CPC:(k + 1) * CPC]
    s1_ref[...] = lax.dot_general(h, a1_ref[...], (((1,), (0,)), ((), ())),
                                  precision=lax.Precision.HIGHEST,
                                  preferred_element_type=F32)
    s2_ref[...] = lax.dot_general(h, a2_ref[...], (((1,), (0,)), ((), ())),
                                  precision=lax.Precision.HIGHEST,
                                  preferred_element_type=F32)


_BA = 1024

_pre_call = pl.pallas_call(
    _pre_body,
    grid=(NP // _BA,),
    in_specs=[
        pl.BlockSpec((_BA, D), lambda i: (i, 0)),
        pl.BlockSpec((D, D), lambda i: (0, 0)),
        pl.BlockSpec((D, 1), lambda i: (0, 0)),
        pl.BlockSpec((D, 1), lambda i: (0, 0)),
    ],
    out_specs=[
        pl.BlockSpec((NSLOT, _BA, CPC), lambda i: (0, i, 0)),
        pl.BlockSpec((_BA, 1), lambda i: (i, 0)),
        pl.BlockSpec((_BA, 1), lambda i: (i, 0)),
    ],
    out_shape=[
        jax.ShapeDtypeStruct((NSLOT, NP, CPC), F32),
        jax.ShapeDtypeStruct((NP, 1), F32),
        jax.ShapeDtypeStruct((NP, 1), F32),
    ],
)


def _edge_body(hstk, s1_hbm, s2_hbm, src_hbm, dst_hbm,
               acc_out, den_out,
               s1_v, s2_v, src_v, dst_v, gidx_v, p_v, rows_v,
               zbuf, zdbuf, acc_sh, den_sh, gsem):
    c = lax.axis_index("c")
    s = lax.axis_index("s")
    r0 = s * RPT

    i16 = lax.iota(jnp.int32, 16)
    rsel = i16 >> 3
    cidx = i16 & 7
    zeros16 = jnp.zeros((16,), F32)

    pltpu.sync_copy(s1_hbm, s1_v)
    pltpu.sync_copy(s2_hbm, s2_v)

    def _zb(j, _):
        plsc.store_scatter(zbuf, [2 * j + rsel, cidx], zeros16)
        return 0
    lax.fori_loop(0, 32, _zb, 0)

    def _zd(j, _):
        zdbuf[pl.ds(j * 16, 16)] = zeros16
        return 0
    lax.fori_loop(0, 28, _zd, 0)

    def _phase(ph, _):
        slot = 2 * ph + c

        def _zacc(k, _2):
            pltpu.sync_copy(zbuf, acc_sh.at[pl.ds(r0 + k * 64, 64), :])
            return 0
        lax.fori_loop(0, RPT // 64, _zacc, 0)

        @pl.when((ph == 0) & (c == 0))
        def _():
            def _zden(k, _2):
                pltpu.sync_copy(zdbuf, den_sh.at[pl.ds(r0 + k * 448, 448)])
                return 0
            lax.fori_loop(0, RPT // 448, _zden, 0)

        plsc.subcore_barrier()

        def _chunk(j, _2):
            base = (s * CPT + j) * CH
            pltpu.sync_copy(src_hbm.at[pl.ds(base, CH)], src_v)
            pltpu.sync_copy(dst_hbm.at[pl.ds(base, CH)], dst_v)
            for v in range(CH // 16):
                sl = pl.ds(v * 16, 16)
                sv = src_v[sl]
                dv = dst_v[sl]
                msk = sv == dv
                dd = jnp.where(msk, jnp.int32(N_NODES), dv)
                ss = jnp.where(msk, jnp.int32(0), sv)
                a = plsc.load_gather(s1_v, [dd]) + plsc.load_gather(s2_v, [ss])
                a = jnp.where(a > 0, a, a * F32(0.2))
                p_v[sl] = jnp.exp(a)
                gidx_v[sl] = ss + slot * NP
                dst_v[sl] = dd
            pltpu.async_copy(hstk.at[gidx_v], rows_v, gsem).wait()

            for k in range(CH // 2):
                ridx = 2 * k + rsel
                pexp = plsc.load_gather(p_v, [ridx])
                vals = plsc.load_gather(rows_v, [ridx, cidx])
                plsc.store_scatter(rows_v, [ridx, cidx], vals * pexp)

            pltpu.sync_copy(rows_v, acc_sh.at[dst_v], add=True)

            @pl.when((ph == 0) & (c == 0))
            def _():
                pltpu.sync_copy(p_v, den_sh.at[dst_v], add=True)
            return 0
        lax.fori_loop(0, CPT, _chunk, 0)

        plsc.subcore_barrier()

        pltpu.sync_copy(acc_sh.at[pl.ds(r0, RPT), :],
                        acc_out.at[slot, pl.ds(r0, RPT), :])
        return 0
    lax.fori_loop(0, NPH, _phase, 0)

    @pl.when(c == 0)
    def _():
        pltpu.sync_copy(den_sh.at[pl.ds(r0, RPT)], den_out.at[pl.ds(r0, RPT)])


_edge_call = pl.kernel(
    _edge_body,
    out_type=[
        jax.ShapeDtypeStruct((NSLOT, NP, CPC), F32),
        jax.ShapeDtypeStruct((NP,), F32),
    ],
    mesh=plsc.VectorSubcoreMesh(core_axis_name="c", subcore_axis_name="s"),
    compiler_params=pltpu.CompilerParams(needs_layout_passes=False,
                                         use_tc_tiling_on_sc=False),
    scratch_types=[
        pltpu.VMEM((NP,), F32),
        pltpu.VMEM((NP,), F32),
        pltpu.VMEM((CH,), jnp.int32),
        pltpu.VMEM((CH,), jnp.int32),
        pltpu.VMEM((CH,), jnp.int32),
        pltpu.VMEM((CH,), F32),
        pltpu.VMEM((CH, CPC), F32),
        pltpu.VMEM((64, CPC), F32),
        pltpu.VMEM((448,), F32),
        pltpu.VMEM_SHARED((NP, CPC), F32),
        pltpu.VMEM_SHARED((NP,), F32),
        pltpu.SemaphoreType.DMA,
    ],
)


def _fin_body(h8_ref, acc_ref, den_ref, s1_ref, s2_ref, b_ref, o_ref):
    sl = s1_ref[...] + s2_ref[...]
    sl = jnp.where(sl > 0, sl, sl * F32(0.2))
    ploop = jnp.exp(sl)
    v = jnp.concatenate(
        [acc_ref[k] + ploop * h8_ref[k] for k in range(NSLOT)], axis=1)
    dent = den_ref[...] + ploop + F32(1e-16)
    v = v / dent + b_ref[...]
    nr = jnp.sqrt(jnp.sum(v * v, axis=1, keepdims=True))
    o_ref[...] = v / jnp.maximum(nr, F32(1e-12))


_BC = 1000

_fin_call = pl.pallas_call(
    _fin_body,
    grid=(N_NODES // _BC,),
    in_specs=[
        pl.BlockSpec((NSLOT, _BC, CPC), lambda i: (0, i, 0)),
        pl.BlockSpec((NSLOT, _BC, CPC), lambda i: (0, i, 0)),
        pl.BlockSpec((_BC, 1), lambda i: (i, 0)),
        pl.BlockSpec((_BC, 1), lambda i: (i, 0)),
        pl.BlockSpec((_BC, 1), lambda i: (i, 0)),
        pl.BlockSpec((1, D), lambda i: (0, 0)),
    ],
    out_specs=pl.BlockSpec((_BC, D), lambda i: (i, 0)),
    out_shape=jax.ShapeDtypeStruct((N_NODES, D), F32),
)


def kernel(x, edge_indices, W, att, bias):
    xp = jnp.pad(x, ((0, NP - N_NODES), (0, 0)))
    a = att.reshape(2 * D)
    a1 = a[:D].reshape(D, 1)
    a2 = a[D:].reshape(D, 1)

    h8, s1, s2 = _pre_call(xp, W, a1, a2)

    acc, den = _edge_call(
        h8.reshape(NSLOT * NP, CPC),
        s1.reshape(NP),
        s2.reshape(NP),
        edge_indices[0],
        edge_indices[1],
    )

    return _fin_call(h8, acc, den.reshape(NP, 1), s1, s2, bias.reshape(1, D))


# speedup vs baseline: 8.9136x; 2.3315x over previous
import jax
import jax.numpy as jnp
from jax import lax
from jax.experimental import pallas as pl
from jax.experimental.pallas import tpu as pltpu
from jax.experimental.pallas import tpu_sc as plsc

F32 = jnp.float32

N_NODES = 50000
N_EDGES = 800000
D = 64
CPC = 8
NSLOT = D // CPC
NPH = NSLOT // 2
NP = 50176
NSUB = 16
RPT = NP // NSUB
CH = 80
CPT = N_EDGES // (CH * NSUB)


def _pre_body(x_ref, w_ref, a1_ref, a2_ref, h8_ref, s1_ref, s2_ref):
    xb = x_ref[...]
    h = lax.dot_general(xb, w_ref[...], (((1,), (0,)), ((), ())),
                        precision=lax.Precision.HIGHEST,
                        preferred_element_type=F32)
    for k in range(NSLOT):
        h8_ref[k] = h[:, k * CPC:(k + 1) * CPC]
    s1_ref[...] = lax.dot_general(h, a1_ref[...], (((1,), (0,)), ((), ())),
                                  precision=lax.Precision.HIGHEST,
                                  preferred_element_type=F32)
    s2_ref[...] = lax.dot_general(h, a2_ref[...], (((1,), (0,)), ((), ())),
                                  precision=lax.Precision.HIGHEST,
                                  preferred_element_type=F32)


_BA = 1024

_pre_call = pl.pallas_call(
    _pre_body,
    grid=(NP // _BA,),
    in_specs=[
        pl.BlockSpec((_BA, D), lambda i: (i, 0)),
        pl.BlockSpec((D, D), lambda i: (0, 0)),
        pl.BlockSpec((D, 1), lambda i: (0, 0)),
        pl.BlockSpec((D, 1), lambda i: (0, 0)),
    ],
    out_specs=[
        pl.BlockSpec((NSLOT, _BA, CPC), lambda i: (0, i, 0)),
        pl.BlockSpec((_BA, 1), lambda i: (i, 0)),
        pl.BlockSpec((_BA, 1), lambda i: (i, 0)),
    ],
    out_shape=[
        jax.ShapeDtypeStruct((NSLOT, NP, CPC), F32),
        jax.ShapeDtypeStruct((NP, 1), F32),
        jax.ShapeDtypeStruct((NP, 1), F32),
    ],
)


NBUF = 4


def _edge_body(hstk, s1_hbm, s2_hbm, src_hbm, dst_hbm,
               acc_out, *scr):
    (s1_v, s2_v) = scr[0:2]
    srcv = scr[2:6]
    dstv = scr[6:10]
    gidxv = scr[10:14]
    sidxv = scr[14:18]
    pv = scr[18:22]
    rowsv = scr[22:26]
    zbuf, acc_sh = scr[26:28]
    isem = scr[28:32]
    gsem = scr[32:36]
    ssem = scr[36:40]

    c = lax.axis_index("c")
    s = lax.axis_index("s")
    r0 = s * RPT

    i16 = lax.iota(jnp.int32, 16)
    rsel = i16 >> 3
    cidx = i16 & 7
    zero16i = jnp.zeros((16,), jnp.int32)
    zeros16 = jnp.zeros((16,), F32)

    pltpu.sync_copy(s1_hbm, s1_v)
    pltpu.sync_copy(s2_hbm, s2_v)

    def _zb(j, _):
        plsc.store_scatter(zbuf, [2 * j + rsel, cidx], zeros16)
        return 0
    lax.fori_loop(0, 32, _zb, 0)

    def _zacc():
        def _z(k, _2):
            pltpu.sync_copy(zbuf, acc_sh.at[pl.ds(r0 + k * 64, 64), :])
            return 0
        lax.fori_loop(0, RPT // 64, _z, 0)

    base0 = s * CPT * CH

    def idx_issue(j, u):
        b = base0 + j * CH
        pltpu.async_copy(src_hbm.at[pl.ds(b, CH)], srcv[u], isem[u])
        pltpu.async_copy(dst_hbm.at[pl.ds(b, CH)], dstv[u], isem[u])

    def idx_wait(u):
        pltpu.make_async_copy(src_hbm.at[pl.ds(0, CH)], srcv[u],
                              isem[u]).wait()
        pltpu.make_async_copy(dst_hbm.at[pl.ds(0, CH)], dstv[u],
                              isem[u]).wait()

    def alpha(u, goff, with_gidx=True):
        for v in range(CH // 16):
            sl = pl.ds(v * 16, 16)
            sv = srcv[u][sl]
            dv = dstv[u][sl]
            msk = sv == dv
            dd = jnp.where(msk, jnp.int32(N_NODES), dv)
            ss = jnp.where(msk, jnp.int32(0), sv)
            a = (plsc.load_gather(s1_v, [dd])
                 + plsc.load_gather(s2_v, [ss]))
            a = jnp.where(a > 0, a, a * F32(0.2))
            pv[u][sl] = jnp.exp(a)
            if with_gidx:
                gidxv[u][sl] = ss + goff
            sidxv[u][sl] = dd

    def gather_issue(u):
        pltpu.async_copy(hstk.at[gidxv[u]], rowsv[u], gsem[u])

    def gather_wait(u):
        pltpu.make_async_copy(hstk.at[gidxv[u]], rowsv[u],
                              gsem[u]).wait()

    def scale(u):
        def _sc(k, _2):
            for d in range(4):
                ridx = 8 * k + 2 * d + rsel
                pexp = plsc.load_gather(pv[u], [ridx])
                vals = plsc.load_gather(rowsv[u], [ridx, cidx])
                plsc.store_scatter(rowsv[u], [ridx, cidx], vals * pexp)
            return 0
        lax.fori_loop(0, CH // 8, _sc, 0)

    def pstore(u):
        for v in range(CH // 16):
            sl = pl.ds(v * 16, 16)
            plsc.store_scatter(rowsv[u], [v * 16 + i16, zero16i],
                               pv[u][sl])

    def scat_issue(u):
        pltpu.async_copy(rowsv[u], acc_sh.at[sidxv[u]], ssem[u],
                         add=True)

    def scat_wait(u):
        pltpu.make_async_copy(rowsv[u], acc_sh.at[sidxv[u]],
                              ssem[u]).wait()

    def run_pipeline(goff, is_den):
        def step(j, u, with_scat_wait=True, with_prefetch=True,
                 with_idx=True):
            if with_scat_wait:
                scat_wait((u + 2) % NBUF)
            if with_prefetch:
                idx_wait((u + 2) % NBUF)
                alpha((u + 2) % NBUF, goff, with_gidx=not is_den)
                if not is_den:
                    gather_issue((u + 2) % NBUF)
            if with_idx:
                idx_issue(j + 3, (u + 3) % NBUF)
            if is_den:
                pstore(u)
            else:
                gather_wait(u)
                scale(u)
            scat_issue(u)

        idx_issue(0, 0)
        idx_issue(1, 1)
        idx_issue(2, 2)
        idx_wait(0)
        alpha(0, goff, with_gidx=not is_den)
        idx_wait(1)
        alpha(1, goff, with_gidx=not is_den)
        if not is_den:
            gather_issue(0)
            gather_issue(1)

        step(0, 0, with_scat_wait=False)
        step(1, 1, with_scat_wait=False)
        step(2, 2)
        step(3, 3)

        def _t(t, _2):
            j = 4 * t
            step(j + 0, 0)
            step(j + 1, 1)
            step(j + 2, 2)
            step(j + 3, 3)
            return 0
        lax.fori_loop(1, (CPT - 5) // 4, _t, 0)

        e0 = CPT - 5
        step(e0 + 0, 0)
        step(e0 + 1, 1)
        step(e0 + 2, 2, with_idx=False)
        step(e0 + 3, 3, with_prefetch=False, with_idx=False)
        step(e0 + 4, 0, with_prefetch=False, with_idx=False)
        scat_wait(3)
        scat_wait(0)

    def _phase(ph, _):
        slot = 2 * ph + c
        _zacc()
        plsc.subcore_barrier()
        run_pipeline(slot * NP, False)
        plsc.subcore_barrier()
        pltpu.sync_copy(acc_sh.at[pl.ds(r0, RPT), :],
                        acc_out.at[slot, pl.ds(r0, RPT), :])
        return 0
    lax.fori_loop(0, NPH, _phase, 0)

    def _zrows(k, _2):
        for _u in range(NBUF):
            plsc.store_scatter(rowsv[_u], [2 * k + rsel, cidx], zeros16)
        return 0
    lax.fori_loop(0, CH // 2, _zrows, 0)
    _zacc()
    plsc.subcore_barrier()
    run_pipeline(0, True)
    plsc.subcore_barrier()

    @pl.when(c == 0)
    def _():
        pltpu.sync_copy(acc_sh.at[pl.ds(r0, RPT), :],
                        acc_out.at[NSLOT, pl.ds(r0, RPT), :])


_edge_call = pl.kernel(
    _edge_body,
    out_type=jax.ShapeDtypeStruct((NSLOT + 1, NP, CPC), F32),
    mesh=plsc.VectorSubcoreMesh(core_axis_name="c", subcore_axis_name="s"),
    compiler_params=pltpu.CompilerParams(needs_layout_passes=False,
                                         use_tc_tiling_on_sc=False),
    scratch_types=(
        [
            pltpu.VMEM((NP,), F32),
            pltpu.VMEM((NP,), F32),
        ]
        + [pltpu.VMEM((CH,), jnp.int32)] * NBUF
        + [pltpu.VMEM((CH,), jnp.int32)] * NBUF
        + [pltpu.VMEM((CH,), jnp.int32)] * NBUF
        + [pltpu.VMEM((CH,), jnp.int32)] * NBUF
        + [pltpu.VMEM((CH,), F32)] * NBUF
        + [pltpu.VMEM((CH, CPC), F32)] * NBUF
        + [
            pltpu.VMEM((64, CPC), F32),
            pltpu.VMEM_SHARED((NP, CPC), F32),
        ]
        + [pltpu.SemaphoreType.DMA] * (3 * NBUF)
    ),
)


def _fin_body(h8_ref, acc_ref, s1_ref, s2_ref, b_ref, o_ref):
    sl = s1_ref[...] + s2_ref[...]
    sl = jnp.where(sl > 0, sl, sl * F32(0.2))
    ploop = jnp.exp(sl)
    v = jnp.concatenate(
        [acc_ref[k] + ploop * h8_ref[k] for k in range(NSLOT)], axis=1)
    dent = acc_ref[NSLOT, :, 0:1] + ploop + F32(1e-16)
    v = v / dent + b_ref[...]
    nr = jnp.sqrt(jnp.sum(v * v, axis=1, keepdims=True))
    o_ref[...] = v / jnp.maximum(nr, F32(1e-12))


_BC = 1000

_fin_call = pl.pallas_call(
    _fin_body,
    grid=(N_NODES // _BC,),
    in_specs=[
        pl.BlockSpec((NSLOT, _BC, CPC), lambda i: (0, i, 0)),
        pl.BlockSpec((NSLOT + 1, _BC, CPC), lambda i: (0, i, 0)),
        pl.BlockSpec((_BC, 1), lambda i: (i, 0)),
        pl.BlockSpec((_BC, 1), lambda i: (i, 0)),
        pl.BlockSpec((1, D), lambda i: (0, 0)),
    ],
    out_specs=pl.BlockSpec((_BC, D), lambda i: (i, 0)),
    out_shape=jax.ShapeDtypeStruct((N_NODES, D), F32),
)


def kernel(x, edge_indices, W, att, bias):
    xp = jnp.pad(x, ((0, NP - N_NODES), (0, 0)))
    a = att.reshape(2 * D)
    a1 = a[:D].reshape(D, 1)
    a2 = a[D:].reshape(D, 1)

    h8, s1, s2 = _pre_call(xp, W, a1, a2)

    acc = _edge_call(
        h8.reshape(NSLOT * NP, CPC),
        s1.reshape(NP),
        s2.reshape(NP),
        edge_indices[0],
        edge_indices[1],
    )

    return _fin_call(h8, acc, s1, s2, bias.reshape(1, D))
